# baseline (speedup 1.0000x reference)
import functools

import jax
import jax.numpy as jnp
from jax.experimental import pallas as pl
from jax.experimental.pallas import tpu as pltpu


def _tsig(x):
    return 0.5 * jnp.tanh(0.5 * x) + 0.5


def _body(q_ref, sup_ref, w1t_ref, b1_ref, w2t_ref, b2_ref, lng_ref, lnb_ref,
          wiht_ref, wh1t_ref, wh2t_ref, bias_ref, out_ref, *, D, H):
    sup = sup_ref[...]
    t = jnp.maximum(jnp.dot(sup, w1t_ref[...],
                            preferred_element_type=jnp.float32) + b1_ref[...], 0.0)
    t = jnp.dot(t, w2t_ref[...],
                preferred_element_type=jnp.float32) + b2_ref[...] + sup
    mu = jnp.mean(t, axis=1, keepdims=True)
    var = jnp.sum((t - mu) * (t - mu), axis=1, keepdims=True) / (D - 1)
    t = (t - mu) / (jnp.sqrt(var) + 1e-3) * lng_ref[...] + lnb_ref[...]
    sg = jnp.mean(t, axis=0, keepdims=True)
    sgc = jnp.dot(sg, wh2t_ref[...],
                  preferred_element_type=jnp.float32)

    q = q_ref[...]
    base = jnp.dot(q, wiht_ref[...],
                   preferred_element_type=jnp.float32) + bias_ref[...]
    base2 = base + sgc

    i = _tsig(base[:, :H])
    g = jnp.tanh(base[:, 2 * H:3 * H])
    o = _tsig(base[:, 3 * H:])
    c = i * g
    h = q + o * jnp.tanh(c[:, :D])

    for _ in range(2):
        gates = base2 + jnp.dot(h, wh1t_ref[...],
                                preferred_element_type=jnp.float32)
        i = _tsig(gates[:, :H])
        f = _tsig(gates[:, H:2 * H])
        g = jnp.tanh(gates[:, 2 * H:3 * H])
        o = _tsig(gates[:, 3 * H:])
        c = f * c + i * g
        h = q + o * jnp.tanh(c[:, :D])

    gates = base2 + jnp.dot(h, wh1t_ref[...],
                            preferred_element_type=jnp.float32)
    i = _tsig(gates[:, :D])
    f = _tsig(gates[:, H:H + D])
    g = jnp.tanh(gates[:, 2 * H:2 * H + D])
    o = _tsig(gates[:, 3 * H:])
    cD = f * c[:, :D] + i * g
    h = q + o * jnp.tanh(cD)

    out_ref[...] = jax.lax.dot_general(
        h, sg, (((1,), (1,)), ((), ())),
        preferred_element_type=jnp.float32)


def kernel(query, support, W1, b1, W2, b2, ln_g, ln_b, W_ih, W_hh, b_ih, b_hh):
    B, D = query.shape
    H = W_hh.shape[1]
    G = 3 * H + D
    Bt = 1024

    w_hhT = W_hh.T
    operands = (
        query,
        support,
        W1.T,
        b1.reshape(1, -1),
        W2.T,
        b2.reshape(1, -1),
        ln_g.reshape(1, -1),
        ln_b.reshape(1, -1),
        W_ih.T[:, :G],
        w_hhT[:D, :G],
        w_hhT[D:, :G],
        (b_ih + b_hh)[:G].reshape(1, -1),
    )

    full = lambda shape: pl.BlockSpec(shape, lambda i: (0, 0))
    in_specs = [
        pl.BlockSpec((Bt, D), lambda i: (i, 0)),
        full(support.shape),
        full((D, 2 * D)),
        full((1, 2 * D)),
        full((2 * D, D)),
        full((1, D)),
        full((1, D)),
        full((1, D)),
        full((D, G)),
        full((D, G)),
        full((D, G)),
        full((1, G)),
    ]

    out = pl.pallas_call(
        functools.partial(_body, D=D, H=H),
        grid=(B // Bt,),
        in_specs=in_specs,
        out_specs=pl.BlockSpec((Bt, 1), lambda i: (i, 0)),
        out_shape=jax.ShapeDtypeStruct((B, 1), jnp.float32),
        compiler_params=pltpu.CompilerParams(
            dimension_semantics=("parallel",),
        ),
    )(*operands)
    return out.reshape(B)


# speedup vs baseline: 1.4839x; 1.4839x over previous
import functools

import jax
import jax.numpy as jnp
from jax.experimental import pallas as pl
from jax.experimental.pallas import tpu as pltpu


def _sigp(y):
    return 0.5 * jnp.tanh(y) + 0.5


def _dotf(a, w_ref):
    return jnp.dot(a, w_ref[...], preferred_element_type=jnp.float32)


def _body(q_ref, sup_ref, w1t_ref, b1_ref, w2t_ref, b2_ref, lng_ref, lnb_ref,
          wih_ref, whh_ref, wsg_ref, bias_ref, out_ref, *, D):
    sup = sup_ref[...]
    t = jnp.maximum(jnp.dot(sup, w1t_ref[...],
                            preferred_element_type=jnp.float32) + b1_ref[...], 0.0)
    t = jnp.dot(t, w2t_ref[...],
                preferred_element_type=jnp.float32) + b2_ref[...] + sup
    mu = jnp.mean(t, axis=1, keepdims=True)
    var = jnp.sum((t - mu) * (t - mu), axis=1, keepdims=True) / (D - 1)
    t = (t - mu) / (jnp.sqrt(var) + 1e-3) * lng_ref[...] + lnb_ref[...]
    sg = jnp.mean(t, axis=0, keepdims=True)
    sgc = jnp.dot(sg, wsg_ref[...],
                  preferred_element_type=jnp.float32)

    q = q_ref[...]
    base = _dotf(q, wih_ref) + (bias_ref[...] + sgc)

    i = _sigp(base[:, :D] - sgc[:, :D])
    g = jnp.tanh(base[:, 2 * D:3 * D] - sgc[:, 2 * D:3 * D])
    o = _sigp(base[:, 3 * D:] - sgc[:, 3 * D:])
    c = i * g
    h = q + o * jnp.tanh(c)

    for _ in range(3):
        gates = base + _dotf(h, whh_ref)
        i = _sigp(gates[:, :D])
        f = _sigp(gates[:, D:2 * D])
        g = jnp.tanh(gates[:, 2 * D:3 * D])
        o = _sigp(gates[:, 3 * D:])
        c = f * c + i * g
        h = q + o * jnp.tanh(c)

    out_ref[...] = jax.lax.dot_general(
        sg, h, (((1,), (1,)), ((), ())),
        preferred_element_type=jnp.float32)


def _live_cols(w, D, H):
    return jnp.concatenate(
        [w[..., 0:D], w[..., H:H + D],
         w[..., 2 * H:2 * H + D], w[..., 3 * H:3 * H + D]], axis=-1)


def kernel(query, support, W1, b1, W2, b2, ln_g, ln_b, W_ih, W_hh, b_ih, b_hh):
    B, D = query.shape
    H = W_hh.shape[1]
    Bt = 4096

    w_hhT = W_hh.T
    scale = jnp.concatenate(
        [jnp.full((2 * D,), 0.5, jnp.float32),
         jnp.ones((D,), jnp.float32),
         jnp.full((D,), 0.5, jnp.float32)])
    operands = (
        query,
        support,
        W1.T,
        b1.reshape(1, -1),
        W2.T,
        b2.reshape(1, -1),
        ln_g.reshape(1, -1),
        ln_b.reshape(1, -1),
        _live_cols(W_ih.T, D, H) * scale,
        _live_cols(w_hhT[:D], D, H) * scale,
        _live_cols(w_hhT[D:], D, H) * scale,
        (_live_cols(b_ih + b_hh, D, H) * scale).reshape(1, -1),
    )

    full = lambda shape: pl.BlockSpec(shape, lambda i: (0, 0))
    in_specs = [
        pl.BlockSpec((Bt, D), lambda i: (i, 0)),
        full(support.shape),
        full((D, 2 * D)),
        full((1, 2 * D)),
        full((2 * D, D)),
        full((1, D)),
        full((1, D)),
        full((1, D)),
        full((D, 4 * D)),
        full((D, 4 * D)),
        full((D, 4 * D)),
        full((1, 4 * D)),
    ]

    out = pl.pallas_call(
        functools.partial(_body, D=D),
        grid=(B // Bt,),
        in_specs=in_specs,
        out_specs=pl.BlockSpec((1, Bt), lambda i: (0, i)),
        out_shape=jax.ShapeDtypeStruct((1, B), jnp.float32),
        compiler_params=pltpu.CompilerParams(
            dimension_semantics=("parallel",),
        ),
    )(*operands)
    return out.reshape(B)
